# baseline (speedup 1.0000x reference)
import functools

import jax
import jax.numpy as jnp
from jax.experimental import pallas as pl
from jax.experimental.pallas import tpu as pltpu


def _ln_lrelu(a):
    mu = jnp.mean(a, axis=1, keepdims=True)
    var = jnp.mean((a - mu) ** 2, axis=1, keepdims=True)
    y = (a - mu) / jnp.sqrt(var)
    return jnp.maximum(0.05 * y, y)


_DN = (((1,), (1,)), ((), ()))


def _nnue_body(x_ref, w1_ref, w2_ref, w3_ref, w4_ref, out_ref,
               acc1, acc2, *, steps):
    k = pl.program_id(0)

    @pl.when(k == 0)
    def _init():
        acc1[...] = jnp.zeros_like(acc1)
        acc2[...] = jnp.zeros_like(acc2)

    w1c = w1_ref[:, k // 2, :].astype(jnp.bfloat16)
    part = jax.lax.dot_general(
        x_ref[...].astype(jnp.bfloat16), w1c, _DN,
        preferred_element_type=jnp.float32)

    @pl.when(k % 2 == 0)
    def _acc_x1():
        acc1[...] += part

    @pl.when(k % 2 == 1)
    def _acc_x2():
        acc2[...] += part

    @pl.when(k == steps - 1)
    def _epilogue():
        hp = jax.lax.Precision.HIGHEST
        h1 = _ln_lrelu(acc1[...])
        h2 = _ln_lrelu(acc2[...])
        h1 = _ln_lrelu(jax.lax.dot_general(
            h1, w2_ref[...], _DN, precision=hp,
            preferred_element_type=jnp.float32))
        h2 = _ln_lrelu(jax.lax.dot_general(
            h2, w2_ref[...], _DN, precision=hp,
            preferred_element_type=jnp.float32))
        h = jnp.concatenate([h1, h2], axis=1)
        h = _ln_lrelu(jax.lax.dot_general(
            h, w3_ref[...], _DN, precision=hp,
            preferred_element_type=jnp.float32))
        out_ref[...] = jax.lax.dot_general(
            h, w4_ref[...], _DN, precision=hp,
            preferred_element_type=jnp.float32)


def kernel(x, W1, W2, W3, W4):
    n_out, features = W1.shape
    batch = x.size // (2 * features)
    x = x.reshape(batch, 2 * features)

    bk = 1024
    k_chunks = features // bk
    steps = 2 * k_chunks
    w1r = W1.reshape(n_out, k_chunks, bk)

    return pl.pallas_call(
        functools.partial(_nnue_body, steps=steps),
        grid=(steps,),
        in_specs=[
            pl.BlockSpec((batch, bk),
                         lambda k, kc=k_chunks: (0, (k % 2) * kc + k // 2)),
            pl.BlockSpec(w1r.shape, lambda k: (0, 0, 0)),
            pl.BlockSpec(W2.shape, lambda k: (0, 0)),
            pl.BlockSpec(W3.shape, lambda k: (0, 0)),
            pl.BlockSpec(W4.shape, lambda k: (0, 0)),
        ],
        out_specs=pl.BlockSpec((batch, 1), lambda k: (0, 0)),
        out_shape=jax.ShapeDtypeStruct((batch, 1), jnp.float32),
        scratch_shapes=[
            pltpu.VMEM((batch, n_out), jnp.float32),
            pltpu.VMEM((batch, n_out), jnp.float32),
        ],
        compiler_params=pltpu.CompilerParams(
            dimension_semantics=("arbitrary",)),
    )(x, w1r, W2, W3, W4)


# speedup vs baseline: 1.6694x; 1.6694x over previous
import functools

import jax
import jax.numpy as jnp
from jax.experimental import pallas as pl
from jax.experimental.pallas import tpu as pltpu


def _ln_lrelu(a):
    mu = jnp.mean(a, axis=1, keepdims=True)
    var = jnp.mean((a - mu) ** 2, axis=1, keepdims=True)
    y = (a - mu) / jnp.sqrt(var)
    return jnp.maximum(0.05 * y, y)


_DN = (((1,), (1,)), ((), ()))


def _nnue_body(x_ref, w1_ref, w2_ref, w3_ref, w4_ref, out_ref,
               acc1, acc2, *, steps):
    k = pl.program_id(0)

    @pl.when(k == 0)
    def _init():
        acc1[...] = jnp.zeros_like(acc1)
        acc2[...] = jnp.zeros_like(acc2)

    col = pl.multiple_of((k // 2) * x_ref.shape[1], x_ref.shape[1])
    w1c = w1_ref[:, pl.ds(col, x_ref.shape[1])].astype(jnp.bfloat16)
    part = jax.lax.dot_general(
        x_ref[...].astype(jnp.bfloat16), w1c, _DN,
        preferred_element_type=jnp.float32)

    @pl.when(k % 2 == 0)
    def _acc_x1():
        acc1[...] += part

    @pl.when(k % 2 == 1)
    def _acc_x2():
        acc2[...] += part

    @pl.when(k == steps - 1)
    def _epilogue():
        hp = jax.lax.Precision.HIGHEST
        h1 = _ln_lrelu(acc1[...])
        h2 = _ln_lrelu(acc2[...])
        h1 = _ln_lrelu(jax.lax.dot_general(
            h1, w2_ref[...], _DN, precision=hp,
            preferred_element_type=jnp.float32))
        h2 = _ln_lrelu(jax.lax.dot_general(
            h2, w2_ref[...], _DN, precision=hp,
            preferred_element_type=jnp.float32))
        h = jnp.concatenate([h1, h2], axis=1)
        h = _ln_lrelu(jax.lax.dot_general(
            h, w3_ref[...], _DN, precision=hp,
            preferred_element_type=jnp.float32))
        out_ref[...] = jax.lax.dot_general(
            h, w4_ref[...], _DN, precision=hp,
            preferred_element_type=jnp.float32)


def kernel(x, W1, W2, W3, W4):
    n_out, features = W1.shape
    batch = x.size // (2 * features)
    x = x.reshape(batch, 2 * features)

    bk = 1024
    k_chunks = features // bk
    steps = 2 * k_chunks

    return pl.pallas_call(
        functools.partial(_nnue_body, steps=steps),
        grid=(steps,),
        in_specs=[
            pl.BlockSpec((batch, bk),
                         lambda k, kc=k_chunks: (0, (k % 2) * kc + k // 2)),
            pl.BlockSpec(W1.shape, lambda k: (0, 0)),
            pl.BlockSpec(W2.shape, lambda k: (0, 0)),
            pl.BlockSpec(W3.shape, lambda k: (0, 0)),
            pl.BlockSpec(W4.shape, lambda k: (0, 0)),
        ],
        out_specs=pl.BlockSpec((batch, 1), lambda k: (0, 0)),
        out_shape=jax.ShapeDtypeStruct((batch, 1), jnp.float32),
        scratch_shapes=[
            pltpu.VMEM((batch, n_out), jnp.float32),
            pltpu.VMEM((batch, n_out), jnp.float32),
        ],
        compiler_params=pltpu.CompilerParams(
            dimension_semantics=("arbitrary",)),
    )(x, W1, W2, W3, W4)


# speedup vs baseline: 2.0477x; 1.2266x over previous
import functools

import jax
import jax.numpy as jnp
from jax.experimental import pallas as pl
from jax.experimental.pallas import tpu as pltpu


def _ln_lrelu(a):
    mu = jnp.mean(a, axis=1, keepdims=True)
    var = jnp.mean((a - mu) ** 2, axis=1, keepdims=True)
    y = (a - mu) / jnp.sqrt(var)
    return jnp.maximum(0.05 * y, y)


_DN = (((1,), (1,)), ((), ()))


def _nnue_body(x1_ref, x2_ref, w1_ref, w2_ref, w3_ref, w4_ref, out_ref,
               acc1, acc2, *, k_blocks):
    k = pl.program_id(0)

    @pl.when(k == 0)
    def _init():
        acc1[...] = jnp.zeros_like(acc1)
        acc2[...] = jnp.zeros_like(acc2)

    w1b = w1_ref[...].astype(jnp.bfloat16)
    acc1[...] += jax.lax.dot_general(
        x1_ref[...].astype(jnp.bfloat16), w1b, _DN,
        preferred_element_type=jnp.float32)
    acc2[...] += jax.lax.dot_general(
        x2_ref[...].astype(jnp.bfloat16), w1b, _DN,
        preferred_element_type=jnp.float32)

    @pl.when(k == k_blocks - 1)
    def _epilogue():
        hp = jax.lax.Precision.HIGHEST
        h1 = _ln_lrelu(acc1[...])
        h2 = _ln_lrelu(acc2[...])
        h1 = _ln_lrelu(jax.lax.dot_general(
            h1, w2_ref[...], _DN, precision=hp,
            preferred_element_type=jnp.float32))
        h2 = _ln_lrelu(jax.lax.dot_general(
            h2, w2_ref[...], _DN, precision=hp,
            preferred_element_type=jnp.float32))
        h = jnp.concatenate([h1, h2], axis=1)
        h = _ln_lrelu(jax.lax.dot_general(
            h, w3_ref[...], _DN, precision=hp,
            preferred_element_type=jnp.float32))
        out_ref[...] = jax.lax.dot_general(
            h, w4_ref[...], _DN, precision=hp,
            preferred_element_type=jnp.float32)


def kernel(x, W1, W2, W3, W4):
    n_out, features = W1.shape
    batch = x.size // (2 * features)
    x = x.reshape(batch, 2 * features)

    bk = 2048
    k_blocks = features // bk

    return pl.pallas_call(
        functools.partial(_nnue_body, k_blocks=k_blocks),
        grid=(k_blocks,),
        in_specs=[
            pl.BlockSpec((batch, bk), lambda k: (0, k)),
            pl.BlockSpec((batch, bk),
                         lambda k, kb=k_blocks: (0, k + kb)),
            pl.BlockSpec((n_out, bk), lambda k: (0, k)),
            pl.BlockSpec(W2.shape, lambda k: (0, 0)),
            pl.BlockSpec(W3.shape, lambda k: (0, 0)),
            pl.BlockSpec(W4.shape, lambda k: (0, 0)),
        ],
        out_specs=pl.BlockSpec((batch, 1), lambda k: (0, 0)),
        out_shape=jax.ShapeDtypeStruct((batch, 1), jnp.float32),
        scratch_shapes=[
            pltpu.VMEM((batch, n_out), jnp.float32),
            pltpu.VMEM((batch, n_out), jnp.float32),
        ],
        compiler_params=pltpu.CompilerParams(
            dimension_semantics=("arbitrary",)),
    )(x, x, W1, W2, W3, W4)
